# baseline (speedup 1.0000x reference)
import functools

import jax
import jax.numpy as jnp
from jax import lax
from jax.experimental import pallas as pl
from jax.experimental.pallas import tpu as pltpu
from jax.experimental.pallas import tpu_sc as plsc

Q = 262144
N = 128
NC = 2
NS = 16
L = 16
NW = NC * NS
PER_W = Q // NW
BLK = 2048
NBLK = PER_W // BLK
GROUPS = BLK // L

_mesh = plsc.VectorSubcoreMesh(core_axis_name="c", subcore_axis_name="s")


@functools.partial(
    pl.kernel,
    mesh=_mesh,
    out_type=[
        jax.ShapeDtypeStruct((Q * 8,), jnp.int32),
        jax.ShapeDtypeStruct((Q * 8,), jnp.float32),
    ],
    scratch_types=[
        pltpu.VMEM((BLK,), jnp.float32),
        pltpu.VMEM((BLK,), jnp.float32),
        pltpu.VMEM((BLK,), jnp.float32),
        pltpu.VMEM((BLK * 8,), jnp.int32),
        pltpu.VMEM((BLK * 8,), jnp.int32),
        pltpu.VMEM((BLK * 8,), jnp.float32),
        pltpu.SemaphoreType.DMA,
    ],
    compiler_params=pltpu.CompilerParams(needs_layout_passes=False),
)
def _sc_lookup(ux, uy, uz, grid, out_ids, out_w, ux_v, uy_v, uz_v, idx_v, ids_v, w_v, sem):
    wid = lax.axis_index("s") * NC + lax.axis_index("c")
    lanes8 = lax.iota(jnp.int32, L) * 8

    for b in range(NBLK):
        p0 = wid * PER_W + b * BLK
        pltpu.sync_copy(ux.at[pl.ds(p0, BLK)], ux_v)
        pltpu.sync_copy(uy.at[pl.ds(p0, BLK)], uy_v)
        pltpu.sync_copy(uz.at[pl.ds(p0, BLK)], uz_v)

        def body(j, carry):
            s = j * L

            def axis(ref):
                u = ref[pl.ds(s, L)]
                u = jnp.minimum(jnp.maximum(u, 0.0), 127.0)
                i = jnp.minimum(u.astype(jnp.int32), 126)
                f = u - i.astype(jnp.float32)
                return i, f

            xi, fx = axis(ux_v)
            yi, fy = axis(uy_v)
            zi, fz = axis(uz_v)

            base = zi * (N * N) + yi * N + xi
            gx = 1.0 - fx
            gy = 1.0 - fy
            gz = 1.0 - fz
            w00 = gz * gy
            w01 = gz * fy
            w10 = fz * gy
            w11 = fz * fy
            weights = (w00 * gx, w00 * fx, w01 * gx, w01 * fx,
                       w10 * gx, w10 * fx, w11 * gx, w11 * fx)
            offs = (0, 1, N, N + 1,
                    N * N, N * N + 1, N * N + N, N * N + N + 1)

            pos = lanes8 + j * (L * 8)
            for c in range(8):
                plsc.store_scatter(idx_v, [pos + c], base + offs[c])
                plsc.store_scatter(w_v, [pos + c], weights[c])
            return carry

        lax.fori_loop(0, GROUPS, body, 0)

        pltpu.async_copy(grid.at[idx_v], ids_v, sem).wait()
        pltpu.sync_copy(ids_v, out_ids.at[pl.ds(p0 * 8, BLK * 8)])
        pltpu.sync_copy(w_v, out_w.at[pl.ds(p0 * 8, BLK * 8)])


def kernel(points, grid_id, bbox_min, bbox_max):
    scale = (N - 1.0) / (bbox_max - bbox_min)
    ut = ((points - bbox_min) * scale).T
    ids_f, w_f = _sc_lookup(ut[0], ut[1], ut[2], grid_id.reshape(-1))
    return ids_f.reshape(Q, 8), w_f.reshape(Q, 8)
